# baseline (speedup 1.0000x reference)
import functools

import jax
import jax.numpy as jnp
from jax import lax
from jax.experimental import pallas as pl
from jax.experimental.pallas import tpu as pltpu

N = 10000
K = 16
C = 128
G = 8
CG = C // G
EPS = 1e-5
NP = 10240
RB = 512
NB = NP // RB


def _gram_body(x_ref, s_ref, fs_ref):
    @pl.when(pl.program_id(0) == 0)
    def _():
        s_ref[...] = jnp.zeros_like(s_ref)
        fs_ref[...] = jnp.zeros_like(fs_ref)

    x = x_ref[...]
    s_ref[...] += lax.dot_general(x, x, (((0,), (0,)), ((), ())),
                                  preferred_element_type=jnp.float32)
    fs_ref[...] += jnp.sum(x, axis=0, keepdims=True)


def _gram(featp):
    blk = 1024
    return pl.pallas_call(
        _gram_body,
        grid=(NP // blk,),
        in_specs=[pl.BlockSpec((blk, C), lambda i: (i, 0))],
        out_specs=[pl.BlockSpec((C, C), lambda i: (0, 0)),
                   pl.BlockSpec((1, C), lambda i: (0, 0))],
        out_shape=[jax.ShapeDtypeStruct((C, C), jnp.float32),
                   jax.ShapeDtypeStruct((1, C), jnp.float32)],
    )(featp)


def _proj_body(x_ref, wq_ref, cq_ref, wk_ref, ck_ref, wv_ref, bv_ref, ww1_ref,
               v_ref, qw_ref, kw_ref):
    x = x_ref[...]
    ww1 = ww1_ref[...]
    q = jax.nn.relu(jnp.dot(x, wq_ref[...], preferred_element_type=jnp.float32)
                    + cq_ref[...])
    qw_ref[...] = jnp.dot(q, ww1, preferred_element_type=jnp.float32)
    k = jax.nn.relu(jnp.dot(x, wk_ref[...], preferred_element_type=jnp.float32)
                    + ck_ref[...])
    kw_ref[...] = jnp.dot(k, ww1, preferred_element_type=jnp.float32)
    v_ref[...] = jnp.dot(x, wv_ref[...], preferred_element_type=jnp.float32) + bv_ref[...]


def _proj(featp, wq, cq, wk, ck, wv, bv, ww1):
    full = lambda *s: pl.BlockSpec(s, lambda i: (0,) * len(s))
    return pl.pallas_call(
        _proj_body,
        grid=(NB,),
        in_specs=[pl.BlockSpec((RB, C), lambda i: (i, 0)),
                  full(C, C), full(1, C), full(C, C), full(1, C),
                  full(C, C), full(1, C), full(C, G)],
        out_specs=[pl.BlockSpec((RB, C), lambda i: (i, 0)),
                   pl.BlockSpec((RB, G), lambda i: (i, 0)),
                   pl.BlockSpec((RB, G), lambda i: (i, 0))],
        out_shape=[jax.ShapeDtypeStruct((NP, C), jnp.float32),
                   jax.ShapeDtypeStruct((NP, G), jnp.float32),
                   jax.ShapeDtypeStruct((NP, G), jnp.float32)],
    )(featp, wq, cq, wk, ck, wv, bv, ww1)


def _h_block(px, py, pz, w1x, w1y, w1z, cp):
    h = (px[:, :, None] * w1x[None, :, :] + py[:, :, None] * w1y[None, :, :]
         + pz[:, :, None] * w1z[None, :, :] + cp[None, :, :])
    return jax.nn.relu(h)


def _xpass_body(px_ref, py_ref, pz_ref, kwg_ref, qw_ref,
                w1x_ref, w1y_ref, w1z_ref, cp_ref, wp21_ref, cw1_ref,
                x_ref, xs_ref, xss_ref):
    @pl.when(pl.program_id(0) == 0)
    def _():
        xs_ref[...] = jnp.zeros_like(xs_ref)
        xss_ref[...] = jnp.zeros_like(xss_ref)

    h = _h_block(px_ref[...], py_ref[...], pz_ref[...],
                 w1x_ref[...], w1y_ref[...], w1z_ref[...], cp_ref[...])
    pebw = lax.dot_general(h, wp21_ref[...], (((2,), (0,)), ((), ())),
                           preferred_element_type=jnp.float32)
    x = kwg_ref[...] - qw_ref[...][:, None, :] + pebw + cw1_ref[...][None]
    x_ref[...] = x
    gid = pl.program_id(0) * RB + lax.broadcasted_iota(jnp.int32, (RB, 1, 1), 0)
    xm = jnp.where(gid < N, x, 0.0)
    xs_ref[...] += jnp.sum(xm, axis=(0, 1))[None]
    xss_ref[...] += jnp.sum(xm * xm, axis=(0, 1))[None]


def _xpass(px, py, pz, kwg, qw, w1x, w1y, w1z, cp, wp21, cw1):
    full = lambda *s: pl.BlockSpec(s, lambda i: (0,) * len(s))
    return pl.pallas_call(
        _xpass_body,
        grid=(NB,),
        in_specs=[pl.BlockSpec((RB, K), lambda i: (i, 0)),
                  pl.BlockSpec((RB, K), lambda i: (i, 0)),
                  pl.BlockSpec((RB, K), lambda i: (i, 0)),
                  pl.BlockSpec((RB, K, G), lambda i: (i, 0, 0)),
                  pl.BlockSpec((RB, G), lambda i: (i, 0)),
                  full(1, C), full(1, C), full(1, C), full(1, C),
                  full(C, G), full(1, G)],
        out_specs=[pl.BlockSpec((RB, K, G), lambda i: (i, 0, 0)),
                   pl.BlockSpec((1, G), lambda i: (0, 0)),
                   pl.BlockSpec((1, G), lambda i: (0, 0))],
        out_shape=[jax.ShapeDtypeStruct((NP, K, G), jnp.float32),
                   jax.ShapeDtypeStruct((1, G), jnp.float32),
                   jax.ShapeDtypeStruct((1, G), jnp.float32)],
    )(px, py, pz, kwg, qw, w1x, w1y, w1z, cp, wp21, cw1)


def _attn_body(x_ref, px_ref, py_ref, pz_ref,
               sw_ref, tw_ref, ww2_ref, bw2_ref,
               w1x_ref, w1y_ref, w1z_ref, cp_ref, wp2_ref, bp2_ref,
               out_ref, wgt_ref):
    x = x_ref[...]
    xt = jax.nn.relu(x * sw_ref[...][None] + tw_ref[...][None])
    wt = lax.dot_general(xt, ww2_ref[...], (((2,), (0,)), ((), ())),
                         preferred_element_type=jnp.float32) + bw2_ref[...][None]
    mx = jnp.max(wt, axis=1, keepdims=True)
    e = jnp.exp(wt - mx)
    wgt = e / jnp.sum(e, axis=1, keepdims=True)
    wgt_ref[...] = wgt

    h = _h_block(px_ref[...], py_ref[...], pz_ref[...],
                 w1x_ref[...], w1y_ref[...], w1z_ref[...], cp_ref[...])
    bp2 = bp2_ref[...]
    for g in range(G):
        wg = wgt[:, :, g]
        ag = jnp.sum(wg[:, :, None] * h, axis=1)
        og = jnp.dot(ag, wp2_ref[:, g * CG:(g + 1) * CG],
                     preferred_element_type=jnp.float32)
        og += jnp.sum(wg, axis=1)[:, None] * bp2[:, g * CG:(g + 1) * CG]
        out_ref[:, g * CG:(g + 1) * CG] = og


def _attn(xarr, px, py, pz, sw, tw, ww2, bw2, w1x, w1y, w1z, cp, wp2, bp2):
    full = lambda *s: pl.BlockSpec(s, lambda i: (0,) * len(s))
    return pl.pallas_call(
        _attn_body,
        grid=(NB,),
        in_specs=[pl.BlockSpec((RB, K, G), lambda i: (i, 0, 0)),
                  pl.BlockSpec((RB, K), lambda i: (i, 0)),
                  pl.BlockSpec((RB, K), lambda i: (i, 0)),
                  pl.BlockSpec((RB, K), lambda i: (i, 0)),
                  full(1, G), full(1, G), full(G, G), full(1, G),
                  full(1, C), full(1, C), full(1, C), full(1, C),
                  full(C, C), full(1, C)],
        out_specs=[pl.BlockSpec((RB, C), lambda i: (i, 0)),
                   pl.BlockSpec((RB, K, G), lambda i: (i, 0, 0))],
        out_shape=[jax.ShapeDtypeStruct((NP, C), jnp.float32),
                   jax.ShapeDtypeStruct((NP, K, G), jnp.float32)],
    )(xarr, px, py, pz, sw, tw, ww2, bw2, w1x, w1y, w1z, cp, wp2, bp2)


def _qk_fold(S, fsum, W, b, g, beta, n):
    sw = fsum @ W
    m = (sw + n * b) / n
    sumsq = jnp.einsum('cj,cd,dj->j', W, S, W) + 2 * b * sw + n * b * b
    var = sumsq / n - m * m
    s = g / jnp.sqrt(var + EPS)
    t = beta - m * s
    return W * s[None, :], (b * s + t)[None, :]


def kernel(feat, coord, reference_index, Wq, bq, gq, betaq, Wk, bk, gk, betak,
           Wv, bv, Wp1, bp1, gp, betap, Wp2, bp2, Ww1, bw1, gw, betaw, Ww2, bw2):
    idx = reference_index
    featp = jnp.pad(feat, ((0, NP - N), (0, 0)))
    coordp = jnp.concatenate([coord, jnp.broadcast_to(coord[0], (NP - N, 3))], 0)
    idxp = jnp.pad(idx, ((0, NP - N), (0, 0)))

    S, fs = _gram(featp)
    fsum = fs[0]
    wq_, cq = _qk_fold(S, fsum, Wq, bq, gq, betaq, N)
    wk_, ck = _qk_fold(S, fsum, Wk, bk, gk, betak, N)

    v, qw, kw = _proj(featp, wq_, cq, wk_, ck, Wv, bv[None], Ww1)

    posg = coordp[idxp]
    pos = posg - coordp[:, None, :]
    px, py, pz = pos[:, :, 0], pos[:, :, 1], pos[:, :, 2]
    kwg = kw[idxp]
    psum = pos.sum((0, 1))
    pouter = jnp.einsum('nsa,nsb->ab', pos, pos)

    NK = N * K
    pw = psum @ Wp1
    m_p = pw / NK + bp1
    sumsq_p = (jnp.einsum('cj,cd,dj->j', Wp1, pouter, Wp1) / NK
               + 2 * bp1 * pw / NK + bp1 * bp1)
    var_p = sumsq_p - m_p * m_p
    sP = gp / jnp.sqrt(var_p + EPS)
    wp1_ = Wp1 * sP[None, :]
    cP = bp1 * sP + (betap - m_p * sP)
    w1x, w1y, w1z = wp1_[0][None], wp1_[1][None], wp1_[2][None]
    wp21 = Wp2 @ Ww1
    cw1 = (bp2 @ Ww1 + bw1)[None]

    xarr, xs, xss = _xpass(px, py, pz, kwg, qw, w1x, w1y, w1z, cP[None],
                           wp21, cw1)
    m_x = xs[0] / NK
    var_x = xss[0] / NK - m_x * m_x
    sW = gw / jnp.sqrt(var_x + EPS)
    tW = betaw - m_x * sW

    out_tc, wgt = _attn(xarr, px, py, pz, sW[None], tW[None], Ww2, bw2[None],
                        w1x, w1y, w1z, cP[None], Wp2, bp2[None])

    vg = v[idxp].reshape(NP, K, G, CG)
    out_sc = jnp.einsum('nsgi,nsg->ngi', vg, wgt).reshape(NP, C)
    return (out_tc + out_sc)[:N]
